# Initial kernel scaffold; baseline (speedup 1.0000x reference)
import jax
import jax.numpy as jnp
from jax.experimental import pallas as pl


def kernel(x, edge_index, edge_weight_list, W_enc, b_enc, W0a, b0a, W0b, b0b, g0, be0, W1a, b1a, W1b, b1b, g1, be1):
    raise NotImplementedError("write your pallas kernel here")


# speedup vs baseline: 6.6336x; 6.6336x over previous
import functools

import jax
import jax.numpy as jnp
from jax import lax
from jax.experimental import pallas as pl
from jax.experimental.pallas import tpu as pltpu
from jax.experimental.pallas import tpu_sc as plsc

N_NODES_ = 10000
N_PAD = 10240
N_EDGES_ = 320000
N_COMS_ = 4
COM_DIM_ = 32
EPS = 1e-5
BLK = 1024
GRID = N_PAD // BLK
NC, NS = 2, 16
NW = NC * NS
EC = 2000
N_CHUNKS = N_EDGES_ // EC
F32 = jnp.float32


def _stage0(xp, W_enc, b_enc_row, W0a):
    def body(x_ref, we_ref, be_ref, wa_ref, enc_ref, pT_ref):
        xb = x_ref[...]
        enc_ref[...] = jnp.dot(xb, we_ref[...],
                               preferred_element_type=F32) + be_ref[...]
        pT_ref[...] = lax.dot_general(wa_ref[...], xb, (((0,), (1,)), ((), ())),
                                      preferred_element_type=F32)

    return pl.pallas_call(
        body,
        grid=(GRID,),
        in_specs=[
            pl.BlockSpec((BLK, 128), lambda i: (i, 0)),
            pl.BlockSpec((128, 128), lambda i: (0, 0)),
            pl.BlockSpec((1, 128), lambda i: (0, 0)),
            pl.BlockSpec((128, COM_DIM_), lambda i: (0, 0)),
        ],
        out_specs=[
            pl.BlockSpec((BLK, 128), lambda i: (i, 0)),
            pl.BlockSpec((COM_DIM_, BLK), lambda i: (0, i)),
        ],
        out_shape=[
            jax.ShapeDtypeStruct((N_PAD, 128), F32),
            jax.ShapeDtypeStruct((COM_DIM_, N_PAD), F32),
        ],
    )(xp, W_enc, b_enc_row, W0a)


def _stage_pre(base_T, agg_T, ba_t, Wb, bb_t, shared_base):
    def body(base_ref, agg_ref, ba_ref, wb_ref, bb_ref, hpre_ref, st_ref):
        i = pl.program_id(0)
        b = base_ref[...]
        if shared_base:
            b = jnp.concatenate([b] * N_COMS_, axis=0)
        pre = jnp.maximum(b + agg_ref[...] + ba_ref[...], 0.0)
        wb = wb_ref[...]
        outs = []
        for k in range(N_COMS_):
            outs.append(lax.dot_general(
                wb, pre[k * COM_DIM_:(k + 1) * COM_DIM_, :],
                (((0,), (0,)), ((), ())), preferred_element_type=F32))
        hpre = jnp.concatenate(outs, axis=0) + bb_ref[...]
        hpre_ref[...] = hpre
        col = i * BLK + lax.broadcasted_iota(jnp.int32, (128, BLK), 1)
        hm = jnp.where(col < N_NODES_, hpre, 0.0)
        s = jnp.sum(hm, axis=1, keepdims=True)
        ss = jnp.sum(hm * hm, axis=1, keepdims=True)

        @pl.when(i == 0)
        def _():
            st_ref[...] = jnp.zeros_like(st_ref)
        st_ref[...] += jnp.concatenate([s, ss], axis=1)

    base_rows = COM_DIM_ if shared_base else 128
    return pl.pallas_call(
        body,
        grid=(GRID,),
        in_specs=[
            pl.BlockSpec((base_rows, BLK), lambda i: (0, i)),
            pl.BlockSpec((128, BLK), lambda i: (0, i)),
            pl.BlockSpec((128, 1), lambda i: (0, 0)),
            pl.BlockSpec((COM_DIM_, COM_DIM_), lambda i: (0, 0)),
            pl.BlockSpec((128, 1), lambda i: (0, 0)),
        ],
        out_specs=[
            pl.BlockSpec((128, BLK), lambda i: (0, i)),
            pl.BlockSpec((128, 2), lambda i: (0, 0)),
        ],
        out_shape=[
            jax.ShapeDtypeStruct((128, N_PAD), F32),
            jax.ShapeDtypeStruct((128, 2), F32),
        ],
    )(base_T, agg_T, ba_t, Wb, bb_t)


def _stage_var(hpre_T, st):
    def body(hpre_ref, st_ref, var_ref):
        i = pl.program_id(0)
        mu = st_ref[...][:, 0:1] / N_NODES_
        col = i * BLK + lax.broadcasted_iota(jnp.int32, (128, BLK), 1)
        dv = jnp.where(col < N_NODES_, hpre_ref[...] - mu, 0.0)
        s = jnp.sum(dv * dv, axis=1, keepdims=True)

        @pl.when(i == 0)
        def _():
            var_ref[...] = jnp.zeros_like(var_ref)
        var_ref[...] += s

    return pl.pallas_call(
        body,
        grid=(GRID,),
        in_specs=[
            pl.BlockSpec((128, BLK), lambda i: (0, i)),
            pl.BlockSpec((128, 2), lambda i: (0, 0)),
        ],
        out_specs=pl.BlockSpec((128, 1), lambda i: (0, 0)),
        out_shape=jax.ShapeDtypeStruct((128, 1), F32),
    )(hpre_T, st)


def _stage_post(hpre_T, st, varsum, g_t, be_t, Wn):
    with_q = Wn is not None

    def body(*refs):
        if with_q:
            (hpre_ref, st_ref, var_ref, g_ref, be_ref, wn_ref,
             out_ref, q_ref) = refs
        else:
            hpre_ref, st_ref, var_ref, g_ref, be_ref, out_ref = refs
        mu = st_ref[...][:, 0:1] / N_NODES_
        var = var_ref[...] / N_NODES_
        rstd = 1.0 / jnp.sqrt(var + EPS)
        h = jnp.maximum((hpre_ref[...] - mu) * rstd * g_ref[...] + be_ref[...],
                        0.0)
        out_ref[...] = h.T
        if with_q:
            wn = wn_ref[...]
            qs = []
            for k in range(N_COMS_):
                qs.append(lax.dot_general(
                    wn, h[k * COM_DIM_:(k + 1) * COM_DIM_, :],
                    (((0,), (0,)), ((), ())), preferred_element_type=F32))
            q_ref[...] = jnp.concatenate(qs, axis=0)

    in_specs = [
        pl.BlockSpec((128, BLK), lambda i: (0, i)),
        pl.BlockSpec((128, 2), lambda i: (0, 0)),
        pl.BlockSpec((128, 1), lambda i: (0, 0)),
        pl.BlockSpec((128, 1), lambda i: (0, 0)),
        pl.BlockSpec((128, 1), lambda i: (0, 0)),
    ]
    out_specs = [pl.BlockSpec((BLK, 128), lambda i: (i, 0))]
    out_shape = [jax.ShapeDtypeStruct((N_PAD, 128), F32)]
    args = [hpre_T, st, varsum, g_t, be_t]
    if with_q:
        in_specs.append(pl.BlockSpec((COM_DIM_, COM_DIM_), lambda i: (0, 0)))
        out_specs.append(pl.BlockSpec((128, BLK), lambda i: (0, i)))
        out_shape.append(jax.ShapeDtypeStruct((128, N_PAD), F32))
        args.append(Wn)
    res = pl.pallas_call(
        body, grid=(GRID,), in_specs=in_specs, out_specs=out_specs,
        out_shape=out_shape)(*args)
    return res if with_q else (res[0], None)


def _make_sc_seg(ntab):
    mesh = plsc.VectorSubcoreMesh(core_axis_name="c", subcore_axis_name="s",
                                  num_cores=NC, num_subcores=NS)
    scratch = (
        [pltpu.VMEM((N_PAD,), F32) for _ in range(ntab)]
        + [pltpu.VMEM((N_PAD,), F32) for _ in range(N_COMS_)]
        + [pltpu.VMEM((EC,), jnp.int32) for _ in range(4)]
        + [pltpu.VMEM((EC,), F32) for _ in range(2 * N_COMS_)]
        + [pltpu.SemaphoreType.DMA, pltpu.SemaphoreType.DMA]
    )

    @functools.partial(
        pl.kernel,
        out_type=jax.ShapeDtypeStruct((128, N_PAD), F32),
        mesh=mesh,
        scratch_types=scratch,
        compiler_params=pltpu.CompilerParams(needs_layout_passes=False),
    )
    def seg(tab_hbm, src_hbm, dst_hbm, w_hbm, agg_hbm, *refs):
        tabs = refs[0:ntab]
        accs = refs[ntab:ntab + 4]
        sbufs = refs[ntab + 4:ntab + 6]
        dbufs = refs[ntab + 6:ntab + 8]
        wb = refs[ntab + 8:ntab + 16]
        wbufs = [wb[0:2], wb[2:4], wb[4:6], wb[6:8]]
        sems = refs[ntab + 16:ntab + 18]

        wid = lax.axis_index("s") * NC + lax.axis_index("c")

        for t in range(ntab):
            row = t * COM_DIM_ + wid if ntab == 4 else wid
            pltpu.sync_copy(tab_hbm.at[row], tabs[t])

        @pl.loop(0, N_PAD // 16)
        def _(i):
            z = jnp.zeros((16,), F32)
            for a in accs:
                a[pl.ds(i * 16, 16)] = z

        def start(g, b):
            base = g * EC
            pltpu.async_copy(src_hbm.at[pl.ds(base, EC)], sbufs[b], sems[b])
            pltpu.async_copy(dst_hbm.at[pl.ds(base, EC)], dbufs[b], sems[b])
            for k in range(N_COMS_):
                pltpu.async_copy(w_hbm.at[pl.ds(k * N_EDGES_ + base, EC)],
                                 wbufs[k][b], sems[b])

        def wait(b):
            pltpu.make_async_copy(src_hbm.at[pl.ds(0, EC)], sbufs[b],
                                  sems[b]).wait()
            pltpu.make_async_copy(dst_hbm.at[pl.ds(0, EC)], dbufs[b],
                                  sems[b]).wait()
            for k in range(N_COMS_):
                pltpu.make_async_copy(w_hbm.at[pl.ds(0, EC)], wbufs[k][b],
                                      sems[b]).wait()

        def process(b):
            sb, db = sbufs[b], dbufs[b]

            @pl.loop(0, EC // 16, unroll=4)
            def _(j):
                off = j * 16
                sidx = sb[pl.ds(off, 16)]
                didx = db[pl.ds(off, 16)]
                if ntab == 1:
                    v = plsc.load_gather(tabs[0], [sidx])
                    for k in range(N_COMS_):
                        wk = wbufs[k][b][pl.ds(off, 16)]
                        plsc.addupdate_scatter(accs[k], [didx], v * wk)
                else:
                    for k in range(N_COMS_):
                        v = plsc.load_gather(tabs[k], [sidx])
                        wk = wbufs[k][b][pl.ds(off, 16)]
                        plsc.addupdate_scatter(accs[k], [didx], v * wk)

        start(0, 0)

        @pl.loop(0, N_CHUNKS, step=2)
        def _(g):
            start(g + 1, 1)
            wait(0)
            process(0)

            @pl.when(g + 2 < N_CHUNKS)
            def _():
                start(g + 2, 0)
            wait(1)
            process(1)

        for k in range(N_COMS_):
            pltpu.sync_copy(accs[k], agg_hbm.at[k * COM_DIM_ + wid])

    return seg


_sc_seg_shared = _make_sc_seg(1)
_sc_seg_perk = _make_sc_seg(4)


def kernel(x, edge_index, edge_weight_list, W_enc, b_enc,
           W0a, b0a, W0b, b0b, g0, be0,
           W1a, b1a, W1b, b1b, g1, be1):
    src = edge_index[0].astype(jnp.int32)
    dst = edge_index[1].astype(jnp.int32)
    wflat = edge_weight_list.astype(F32).reshape(-1)
    xp = jnp.pad(x.astype(F32), ((0, N_PAD - N_NODES_), (0, 0)))

    enc_p, pT = _stage0(xp, W_enc, b_enc.reshape(1, -1), W0a)

    agg0 = _sc_seg_shared(pT, src, dst, wflat)
    h1pre, st1 = _stage_pre(pT, agg0, jnp.tile(b0a, N_COMS_).reshape(-1, 1),
                            W0b, jnp.tile(b0b, N_COMS_).reshape(-1, 1),
                            shared_base=True)
    vs1 = _stage_var(h1pre, st1)
    out1_p, qT = _stage_post(h1pre, st1, vs1,
                             jnp.tile(g0, N_COMS_).reshape(-1, 1),
                             jnp.tile(be0, N_COMS_).reshape(-1, 1), W1a)

    agg1 = _sc_seg_perk(qT, src, dst, wflat)
    h2pre, st2 = _stage_pre(qT, agg1, jnp.tile(b1a, N_COMS_).reshape(-1, 1),
                            W1b, jnp.tile(b1b, N_COMS_).reshape(-1, 1),
                            shared_base=False)
    vs2 = _stage_var(h2pre, st2)
    out2_p, _ = _stage_post(h2pre, st2, vs2,
                            jnp.tile(g1, N_COMS_).reshape(-1, 1),
                            jnp.tile(be1, N_COMS_).reshape(-1, 1), None)

    return (enc_p[:N_NODES_], out1_p[:N_NODES_], out2_p[:N_NODES_])


# speedup vs baseline: 13.6765x; 2.0617x over previous
import functools

import jax
import jax.numpy as jnp
from jax import lax
from jax.experimental import pallas as pl
from jax.experimental.pallas import tpu as pltpu
from jax.experimental.pallas import tpu_sc as plsc

N_NODES_ = 10000
N_PAD = 10240
N_EDGES_ = 320000
N_COMS_ = 4
COM_DIM_ = 32
EPS = 1e-5
BLK = 1024
GRID = N_PAD // BLK
NC, NS = 2, 16
NW = NC * NS
EC = 2000
N_CHUNKS = N_EDGES_ // EC
F32 = jnp.float32


def _stage0(xp, W_enc, b_enc_row, W0a):
    def body(x_ref, we_ref, be_ref, wa_ref, enc_ref, pT_ref):
        xb = x_ref[...]
        enc_ref[...] = jnp.dot(xb, we_ref[...],
                               preferred_element_type=F32) + be_ref[...]
        pT_ref[...] = lax.dot_general(wa_ref[...], xb, (((0,), (1,)), ((), ())),
                                      preferred_element_type=F32)

    return pl.pallas_call(
        body,
        grid=(GRID,),
        in_specs=[
            pl.BlockSpec((BLK, 128), lambda i: (i, 0)),
            pl.BlockSpec((128, 128), lambda i: (0, 0)),
            pl.BlockSpec((1, 128), lambda i: (0, 0)),
            pl.BlockSpec((128, COM_DIM_), lambda i: (0, 0)),
        ],
        out_specs=[
            pl.BlockSpec((BLK, 128), lambda i: (i, 0)),
            pl.BlockSpec((COM_DIM_, BLK), lambda i: (0, i)),
        ],
        out_shape=[
            jax.ShapeDtypeStruct((N_PAD, 128), F32),
            jax.ShapeDtypeStruct((COM_DIM_, N_PAD), F32),
        ],
    )(xp, W_enc, b_enc_row, W0a)


def _stage_pre(base_T, agg_T, ba_t, Wb, bb_t, shared_base):
    def body(base_ref, agg_ref, ba_ref, wb_ref, bb_ref, hpre_ref, st_ref):
        i = pl.program_id(0)
        b = base_ref[...]
        if shared_base:
            b = jnp.concatenate([b] * N_COMS_, axis=0)
        pre = jnp.maximum(b + agg_ref[...] + ba_ref[...], 0.0)
        wb = wb_ref[...]
        outs = []
        for k in range(N_COMS_):
            outs.append(lax.dot_general(
                wb, pre[k * COM_DIM_:(k + 1) * COM_DIM_, :],
                (((0,), (0,)), ((), ())), preferred_element_type=F32))
        hpre = jnp.concatenate(outs, axis=0) + bb_ref[...]
        hpre_ref[...] = hpre
        col = i * BLK + lax.broadcasted_iota(jnp.int32, (128, BLK), 1)
        hm = jnp.where(col < N_NODES_, hpre, 0.0)
        s = jnp.sum(hm, axis=1, keepdims=True)
        ss = jnp.sum(hm * hm, axis=1, keepdims=True)

        @pl.when(i == 0)
        def _():
            st_ref[...] = jnp.zeros_like(st_ref)
        st_ref[...] += jnp.concatenate([s, ss], axis=1)

    base_rows = COM_DIM_ if shared_base else 128
    return pl.pallas_call(
        body,
        grid=(GRID,),
        in_specs=[
            pl.BlockSpec((base_rows, BLK), lambda i: (0, i)),
            pl.BlockSpec((128, BLK), lambda i: (0, i)),
            pl.BlockSpec((128, 1), lambda i: (0, 0)),
            pl.BlockSpec((COM_DIM_, COM_DIM_), lambda i: (0, 0)),
            pl.BlockSpec((128, 1), lambda i: (0, 0)),
        ],
        out_specs=[
            pl.BlockSpec((128, BLK), lambda i: (0, i)),
            pl.BlockSpec((128, 2), lambda i: (0, 0)),
        ],
        out_shape=[
            jax.ShapeDtypeStruct((128, N_PAD), F32),
            jax.ShapeDtypeStruct((128, 2), F32),
        ],
    )(base_T, agg_T, ba_t, Wb, bb_t)


def _stage_var(hpre_T, st):
    def body(hpre_ref, st_ref, var_ref):
        i = pl.program_id(0)
        mu = st_ref[...][:, 0:1] / N_NODES_
        col = i * BLK + lax.broadcasted_iota(jnp.int32, (128, BLK), 1)
        dv = jnp.where(col < N_NODES_, hpre_ref[...] - mu, 0.0)
        s = jnp.sum(dv * dv, axis=1, keepdims=True)

        @pl.when(i == 0)
        def _():
            var_ref[...] = jnp.zeros_like(var_ref)
        var_ref[...] += s

    return pl.pallas_call(
        body,
        grid=(GRID,),
        in_specs=[
            pl.BlockSpec((128, BLK), lambda i: (0, i)),
            pl.BlockSpec((128, 2), lambda i: (0, 0)),
        ],
        out_specs=pl.BlockSpec((128, 1), lambda i: (0, 0)),
        out_shape=jax.ShapeDtypeStruct((128, 1), F32),
    )(hpre_T, st)


def _stage_post(hpre_T, st, varsum, g_t, be_t, Wn):
    with_q = Wn is not None

    def body(*refs):
        if with_q:
            (hpre_ref, st_ref, var_ref, g_ref, be_ref, wn_ref,
             out_ref, q_ref) = refs
        else:
            hpre_ref, st_ref, var_ref, g_ref, be_ref, out_ref = refs
        mu = st_ref[...][:, 0:1] / N_NODES_
        var = var_ref[...] / N_NODES_
        rstd = 1.0 / jnp.sqrt(var + EPS)
        h = jnp.maximum((hpre_ref[...] - mu) * rstd * g_ref[...] + be_ref[...],
                        0.0)
        out_ref[...] = h.T
        if with_q:
            wn = wn_ref[...]
            qs = []
            for k in range(N_COMS_):
                qs.append(lax.dot_general(
                    wn, h[k * COM_DIM_:(k + 1) * COM_DIM_, :],
                    (((0,), (0,)), ((), ())), preferred_element_type=F32))
            q_ref[...] = jnp.concatenate(qs, axis=0)

    in_specs = [
        pl.BlockSpec((128, BLK), lambda i: (0, i)),
        pl.BlockSpec((128, 2), lambda i: (0, 0)),
        pl.BlockSpec((128, 1), lambda i: (0, 0)),
        pl.BlockSpec((128, 1), lambda i: (0, 0)),
        pl.BlockSpec((128, 1), lambda i: (0, 0)),
    ]
    out_specs = [pl.BlockSpec((BLK, 128), lambda i: (i, 0))]
    out_shape = [jax.ShapeDtypeStruct((N_PAD, 128), F32)]
    args = [hpre_T, st, varsum, g_t, be_t]
    if with_q:
        in_specs.append(pl.BlockSpec((COM_DIM_, COM_DIM_), lambda i: (0, 0)))
        out_specs.append(pl.BlockSpec((128, BLK), lambda i: (0, i)))
        out_shape.append(jax.ShapeDtypeStruct((128, N_PAD), F32))
        args.append(Wn)
    res = pl.pallas_call(
        body, grid=(GRID,), in_specs=in_specs, out_specs=out_specs,
        out_shape=out_shape)(*args)
    return res if with_q else (res[0], None)


def _make_sc_seg(ntab):
    mesh = plsc.VectorSubcoreMesh(core_axis_name="c", subcore_axis_name="s",
                                  num_cores=NC, num_subcores=NS)
    scratch = (
        [pltpu.VMEM((N_PAD,), F32) for _ in range(ntab)]
        + [pltpu.VMEM((N_PAD,), F32) for _ in range(N_COMS_)]
        + [pltpu.VMEM((EC,), jnp.int32) for _ in range(4)]
        + [pltpu.VMEM((EC,), F32) for _ in range(2 * N_COMS_)]
        + [pltpu.SemaphoreType.DMA, pltpu.SemaphoreType.DMA]
    )

    @functools.partial(
        pl.kernel,
        out_type=jax.ShapeDtypeStruct((128, N_PAD), F32),
        mesh=mesh,
        scratch_types=scratch,
        compiler_params=pltpu.CompilerParams(needs_layout_passes=False),
    )
    def seg(tab_hbm, src_hbm, dst_hbm, w_hbm, agg_hbm, *refs):
        tabs = refs[0:ntab]
        accs = refs[ntab:ntab + 4]
        sbufs = refs[ntab + 4:ntab + 6]
        dbufs = refs[ntab + 6:ntab + 8]
        wb = refs[ntab + 8:ntab + 16]
        wbufs = [wb[0:2], wb[2:4], wb[4:6], wb[6:8]]
        sems = refs[ntab + 16:ntab + 18]

        wid = lax.axis_index("s") * NC + lax.axis_index("c")

        for t in range(ntab):
            row = t * COM_DIM_ + wid if ntab == 4 else wid
            pltpu.sync_copy(tab_hbm.at[row], tabs[t])

        @pl.loop(0, N_PAD // 16)
        def _(i):
            z = jnp.zeros((16,), F32)
            for a in accs:
                a[pl.ds(i * 16, 16)] = z

        def start(g, b):
            base = g * EC
            pltpu.async_copy(src_hbm.at[pl.ds(base, EC)], sbufs[b], sems[b])
            pltpu.async_copy(dst_hbm.at[pl.ds(base, EC)], dbufs[b], sems[b])
            for k in range(N_COMS_):
                pltpu.async_copy(w_hbm.at[pl.ds(k * N_EDGES_ + base, EC)],
                                 wbufs[k][b], sems[b])

        def wait(b):
            pltpu.make_async_copy(src_hbm.at[pl.ds(0, EC)], sbufs[b],
                                  sems[b]).wait()
            pltpu.make_async_copy(dst_hbm.at[pl.ds(0, EC)], dbufs[b],
                                  sems[b]).wait()
            for k in range(N_COMS_):
                pltpu.make_async_copy(w_hbm.at[pl.ds(0, EC)], wbufs[k][b],
                                      sems[b]).wait()

        def process(b):
            sb, db = sbufs[b], dbufs[b]

            @plsc.parallel_loop(0, EC // 16, unroll=8)
            def _(j):
                off = j * 16
                sidx = sb[pl.ds(off, 16)]
                didx = db[pl.ds(off, 16)]
                if ntab == 1:
                    v = plsc.load_gather(tabs[0], [sidx])
                    for k in range(N_COMS_):
                        wk = wbufs[k][b][pl.ds(off, 16)]
                        plsc.addupdate_scatter(accs[k], [didx], v * wk)
                else:
                    for k in range(N_COMS_):
                        v = plsc.load_gather(tabs[k], [sidx])
                        wk = wbufs[k][b][pl.ds(off, 16)]
                        plsc.addupdate_scatter(accs[k], [didx], v * wk)

        start(0, 0)

        @pl.loop(0, N_CHUNKS, step=2)
        def _(g):
            start(g + 1, 1)
            wait(0)
            process(0)

            @pl.when(g + 2 < N_CHUNKS)
            def _():
                start(g + 2, 0)
            wait(1)
            process(1)

        for k in range(N_COMS_):
            pltpu.sync_copy(accs[k], agg_hbm.at[k * COM_DIM_ + wid])

    return seg


_sc_seg_shared = _make_sc_seg(1)
_sc_seg_perk = _make_sc_seg(4)


def kernel(x, edge_index, edge_weight_list, W_enc, b_enc,
           W0a, b0a, W0b, b0b, g0, be0,
           W1a, b1a, W1b, b1b, g1, be1):
    src = edge_index[0].astype(jnp.int32)
    dst = edge_index[1].astype(jnp.int32)
    wflat = edge_weight_list.astype(F32).reshape(-1)
    xp = jnp.pad(x.astype(F32), ((0, N_PAD - N_NODES_), (0, 0)))

    enc_p, pT = _stage0(xp, W_enc, b_enc.reshape(1, -1), W0a)

    agg0 = _sc_seg_shared(pT, src, dst, wflat)
    h1pre, st1 = _stage_pre(pT, agg0, jnp.tile(b0a, N_COMS_).reshape(-1, 1),
                            W0b, jnp.tile(b0b, N_COMS_).reshape(-1, 1),
                            shared_base=True)
    vs1 = _stage_var(h1pre, st1)
    out1_p, qT = _stage_post(h1pre, st1, vs1,
                             jnp.tile(g0, N_COMS_).reshape(-1, 1),
                             jnp.tile(be0, N_COMS_).reshape(-1, 1), W1a)

    agg1 = _sc_seg_perk(qT, src, dst, wflat)
    h2pre, st2 = _stage_pre(qT, agg1, jnp.tile(b1a, N_COMS_).reshape(-1, 1),
                            W1b, jnp.tile(b1b, N_COMS_).reshape(-1, 1),
                            shared_base=False)
    vs2 = _stage_var(h2pre, st2)
    out2_p, _ = _stage_post(h2pre, st2, vs2,
                            jnp.tile(g1, N_COMS_).reshape(-1, 1),
                            jnp.tile(be1, N_COMS_).reshape(-1, 1), None)

    return (enc_p[:N_NODES_], out1_p[:N_NODES_], out2_p[:N_NODES_])
